# baseline (speedup 1.0000x reference)
import functools

import jax
import jax.numpy as jnp
from jax import lax
from jax.experimental import pallas as pl
from jax.experimental.pallas import tpu as pltpu
from jax.experimental.pallas import tpu_sc as plsc

B = 4096
D = 8192
L = 16
NC = 2
NS = 16
NW = NC * NS
B_SC = 2048
B_TC = B - B_SC
BPW = B_SC // NW
NSLOT = 4
UNROLL = 4
TCR = 8

_mesh = plsc.VectorSubcoreMesh(core_axis_name="c", subcore_axis_name="s")


def _row_dot(a_ref, w_ref):
    def inner(j, accs):
        base = j * (4 * UNROLL * L)
        new = []
        for q in range(4):
            acc = accs[q]
            for u in range(UNROLL):
                off = base + (q * UNROLL + u) * L
                acc = acc + a_ref[0, pl.ds(off, L)] * w_ref[0, pl.ds(off, L)]
            new.append(acc)
        return tuple(new)

    zeros = jnp.zeros((L,), jnp.float32)
    accs = lax.fori_loop(0, D // (4 * UNROLL * L), inner,
                         (zeros, zeros, zeros, zeros))
    return (accs[0] + accs[1]) + (accs[2] + accs[3])


@functools.partial(
    pl.kernel,
    mesh=_mesh,
    out_type=jax.ShapeDtypeStruct((B_SC,), jnp.float32),
    compiler_params=pltpu.CompilerParams(needs_layout_passes=False),
    scratch_types=[
        pltpu.VMEM((BPW, 1), jnp.int32),
        pltpu.VMEM((BPW, 1), jnp.int32),
        [pltpu.VMEM((1, D), jnp.float32) for _ in range(NSLOT)],
        [pltpu.VMEM((1, D), jnp.float32) for _ in range(NSLOT)],
        pltpu.VMEM((BPW,), jnp.float32),
        pltpu.VMEM((L, L), jnp.float32),
        [pltpu.SemaphoreType.DMA for _ in range(NSLOT)],
        [pltpu.SemaphoreType.DMA for _ in range(NSLOT)],
    ],
)
def _slim_sc(user_hbm, item_hbm, a_hbm, w_hbm, out_hbm,
             uidx, iidx, a_bufs, w_bufs, res, acc_buf, sems_a, sems_w):
    wid = lax.axis_index("s") * NC + lax.axis_index("c")
    pltpu.sync_copy(user_hbm.at[pl.ds(wid * BPW, BPW)], uidx)
    pltpu.sync_copy(item_hbm.at[pl.ds(wid * BPW, BPW)], iidx)

    lane_iota = lax.iota(jnp.int32, L)

    def start(e, slot):
        pltpu.async_copy(a_hbm.at[uidx.at[e]], a_bufs[slot], sems_a[slot])
        pltpu.async_copy(w_hbm.at[iidx.at[e]], w_bufs[slot], sems_w[slot])

    def wait(e, slot):
        pltpu.make_async_copy(
            a_hbm.at[uidx.at[e]], a_bufs[slot], sems_a[slot]).wait()
        pltpu.make_async_copy(
            w_hbm.at[iidx.at[e]], w_bufs[slot], sems_w[slot]).wait()

    for s in range(NSLOT):
        start(s, s)

    def group_body(g, carry):
        for c in range(L):
            slot = c % NSLOT
            e = g * L + c
            wait(e, slot)
            acc_buf[c] = _row_dot(a_bufs[slot], w_bufs[slot])

            @pl.when(e + NSLOT < BPW)
            def _():
                start(e + NSLOT, slot)

        totals = jnp.zeros((L,), jnp.float32)
        for c in range(L):
            col_idx = jnp.full((L,), c, jnp.int32)
            totals = totals + plsc.load_gather(acc_buf, [lane_iota, col_idx])
        res[pl.ds(g * L, L)] = totals
        return carry

    lax.fori_loop(0, BPW // L, group_body, 0)
    pltpu.sync_copy(res, out_hbm.at[pl.ds(wid * BPW, BPW)])


def _tc_body(uref, iref, *refs):
    out_ref = refs[-1]
    a_refs = refs[:TCR]
    w_refs = refs[TCR:2 * TCR]
    a = jnp.concatenate([r[0] for r in a_refs], axis=0)
    w = jnp.concatenate([r[0] for r in w_refs], axis=0)
    out_ref[0, 0, :] = jnp.sum(a * w, axis=1)


def _a_map(k, i, u, it):
    return (u[i * TCR + k], 0, 0)


def _w_map(k, i, u, it):
    return (it[i * TCR + k], 0, 0)


_tc_grid = pltpu.PrefetchScalarGridSpec(
    num_scalar_prefetch=2,
    grid=(B_TC // TCR,),
    in_specs=(
        [pl.BlockSpec((1, 1, D), functools.partial(_a_map, k))
         for k in range(TCR)]
        + [pl.BlockSpec((1, 1, D), functools.partial(_w_map, k))
           for k in range(TCR)]
    ),
    out_specs=pl.BlockSpec((1, 1, TCR), lambda i, u, it: (i, 0, 0)),
)

_tc_call = pl.pallas_call(
    _tc_body,
    grid_spec=_tc_grid,
    out_shape=jax.ShapeDtypeStruct((B_TC // TCR, 1, TCR), jnp.float32),
)


def kernel(user, item, A, W):
    user = user.astype(jnp.int32)
    item = item.astype(jnp.int32)
    out_sc = _slim_sc(user[:B_SC].reshape(B_SC, 1),
                      item[:B_SC].reshape(B_SC, 1), A, W)
    args = [A.reshape(-1, 1, D)] * TCR + [W.reshape(-1, 1, D)] * TCR
    out_tc = _tc_call(user[B_SC:], item[B_SC:], *args).reshape(B_TC)
    return jnp.concatenate([out_sc, out_tc])


# speedup vs baseline: 2.1405x; 2.1405x over previous
import functools

import jax
import jax.numpy as jnp
from jax import lax
from jax.experimental import pallas as pl
from jax.experimental.pallas import tpu as pltpu
from jax.experimental.pallas import tpu_sc as plsc

B = 4096
D = 8192
L = 16
NC = 2
NS = 16
NW = NC * NS
B_SC = 2048
B_TC = B - B_SC
BPW = B_SC // NW
NSLOT = 4
UNROLL = 4
NBUF = 8
STRIP = 128

_mesh = plsc.VectorSubcoreMesh(core_axis_name="c", subcore_axis_name="s")


def _row_dot(a_ref, w_ref):
    def inner(j, accs):
        base = j * (4 * UNROLL * L)
        new = []
        for q in range(4):
            acc = accs[q]
            for u in range(UNROLL):
                off = base + (q * UNROLL + u) * L
                acc = acc + a_ref[0, pl.ds(off, L)] * w_ref[0, pl.ds(off, L)]
            new.append(acc)
        return tuple(new)

    zeros = jnp.zeros((L,), jnp.float32)
    accs = lax.fori_loop(0, D // (4 * UNROLL * L), inner,
                         (zeros, zeros, zeros, zeros))
    return (accs[0] + accs[1]) + (accs[2] + accs[3])


@functools.partial(
    pl.kernel,
    mesh=_mesh,
    out_type=jax.ShapeDtypeStruct((B_SC,), jnp.float32),
    compiler_params=pltpu.CompilerParams(needs_layout_passes=False),
    scratch_types=[
        pltpu.VMEM((BPW, 1), jnp.int32),
        pltpu.VMEM((BPW, 1), jnp.int32),
        [pltpu.VMEM((1, D), jnp.float32) for _ in range(NSLOT)],
        [pltpu.VMEM((1, D), jnp.float32) for _ in range(NSLOT)],
        pltpu.VMEM((BPW,), jnp.float32),
        pltpu.VMEM((L, L), jnp.float32),
        [pltpu.SemaphoreType.DMA for _ in range(NSLOT)],
        [pltpu.SemaphoreType.DMA for _ in range(NSLOT)],
    ],
)
def _slim_sc(user_hbm, item_hbm, a_hbm, w_hbm, out_hbm,
             uidx, iidx, a_bufs, w_bufs, res, acc_buf, sems_a, sems_w):
    wid = lax.axis_index("s") * NC + lax.axis_index("c")
    pltpu.sync_copy(user_hbm.at[pl.ds(wid * BPW, BPW)], uidx)
    pltpu.sync_copy(item_hbm.at[pl.ds(wid * BPW, BPW)], iidx)

    lane_iota = lax.iota(jnp.int32, L)

    def start(e, slot):
        pltpu.async_copy(a_hbm.at[uidx.at[e]], a_bufs[slot], sems_a[slot])
        pltpu.async_copy(w_hbm.at[iidx.at[e]], w_bufs[slot], sems_w[slot])

    def wait(e, slot):
        pltpu.make_async_copy(
            a_hbm.at[uidx.at[e]], a_bufs[slot], sems_a[slot]).wait()
        pltpu.make_async_copy(
            w_hbm.at[iidx.at[e]], w_bufs[slot], sems_w[slot]).wait()

    for s in range(NSLOT):
        start(s, s)

    def group_body(g, carry):
        for c in range(L):
            slot = c % NSLOT
            e = g * L + c
            wait(e, slot)
            acc_buf[c] = _row_dot(a_bufs[slot], w_bufs[slot])

            @pl.when(e + NSLOT < BPW)
            def _():
                start(e + NSLOT, slot)

        totals = jnp.zeros((L,), jnp.float32)
        for c in range(L):
            col_idx = jnp.full((L,), c, jnp.int32)
            totals = totals + plsc.load_gather(acc_buf, [lane_iota, col_idx])
        res[pl.ds(g * L, L)] = totals
        return carry

    lax.fori_loop(0, BPW // L, group_body, 0)
    pltpu.sync_copy(res, out_hbm.at[pl.ds(wid * BPW, BPW)])


def _tc_body(uref, iref, a_hbm, w_hbm, out_ref, a_scr, w_scr, sems):
    strip_iota = lax.broadcasted_iota(jnp.int32, (STRIP,), 0)

    def start(e, s):
        pltpu.make_async_copy(
            a_hbm.at[uref[e]], a_scr.at[s], sems.at[0, s]).start()
        pltpu.make_async_copy(
            w_hbm.at[iref[e]], w_scr.at[s], sems.at[1, s]).start()

    def wait(e, s):
        pltpu.make_async_copy(
            a_hbm.at[uref[e]], a_scr.at[s], sems.at[0, s]).wait()
        pltpu.make_async_copy(
            w_hbm.at[iref[e]], w_scr.at[s], sems.at[1, s]).wait()

    for s in range(NBUF):
        start(s, s)

    gp_per_strip = STRIP // NBUF

    def group_body(g, acc):
        for s in range(NBUF):
            e = g * NBUF + s
            wait(e, s)
            prod = jnp.sum(a_scr[s] * w_scr[s])
            k = (g % gp_per_strip) * NBUF + s
            acc = jnp.where(strip_iota == k, prod, acc)

            @pl.when(e + NBUF < B_TC)
            def _():
                start(e + NBUF, s)

        @pl.when(g % gp_per_strip == gp_per_strip - 1)
        def _():
            out_ref[pl.ds((g // gp_per_strip) * STRIP, STRIP)] = acc

        return jnp.where(g % gp_per_strip == gp_per_strip - 1,
                         jnp.zeros((STRIP,), jnp.float32), acc)

    lax.fori_loop(0, B_TC // NBUF, group_body,
                  jnp.zeros((STRIP,), jnp.float32))


_tc_call = pl.pallas_call(
    _tc_body,
    grid_spec=pltpu.PrefetchScalarGridSpec(
        num_scalar_prefetch=2,
        grid=(1,),
        in_specs=[
            pl.BlockSpec(memory_space=pl.ANY),
            pl.BlockSpec(memory_space=pl.ANY),
        ],
        out_specs=pl.BlockSpec(memory_space=pltpu.MemorySpace.VMEM),
        scratch_shapes=[
            pltpu.VMEM((NBUF, D), jnp.float32),
            pltpu.VMEM((NBUF, D), jnp.float32),
            pltpu.SemaphoreType.DMA((2, NBUF)),
        ],
    ),
    out_shape=jax.ShapeDtypeStruct((B_TC,), jnp.float32),
)


def kernel(user, item, A, W):
    user = user.astype(jnp.int32)
    item = item.astype(jnp.int32)
    out_sc = _slim_sc(user[:B_SC].reshape(B_SC, 1),
                      item[:B_SC].reshape(B_SC, 1), A, W)
    out_tc = _tc_call(user[B_SC:], item[B_SC:], A, W)
    return jnp.concatenate([out_sc, out_tc])


# speedup vs baseline: 6.9753x; 3.2588x over previous
import functools

import jax
import jax.numpy as jnp
from jax import lax
from jax.experimental import pallas as pl
from jax.experimental.pallas import tpu as pltpu
from jax.experimental.pallas import tpu_sc as plsc

B = 4096
D = 8192
L = 16
NC = 2
NS = 16
NW = NC * NS
B_SC = 3584
B_TC = B - B_SC
BPW = B_SC // NW
NSLOT = 4
UNROLL = 4
NBUF = 8
STRIP = 128

_mesh = plsc.VectorSubcoreMesh(core_axis_name="c", subcore_axis_name="s")


def _row_dot(a_ref, w_ref):
    def inner(j, accs):
        base = j * (4 * UNROLL * L)
        new = []
        for q in range(4):
            acc = accs[q]
            for u in range(UNROLL):
                off = base + (q * UNROLL + u) * L
                acc = acc + a_ref[0, pl.ds(off, L)] * w_ref[0, pl.ds(off, L)]
            new.append(acc)
        return tuple(new)

    zeros = jnp.zeros((L,), jnp.float32)
    accs = lax.fori_loop(0, D // (4 * UNROLL * L), inner,
                         (zeros, zeros, zeros, zeros))
    return (accs[0] + accs[1]) + (accs[2] + accs[3])


@functools.partial(
    pl.kernel,
    mesh=_mesh,
    out_type=jax.ShapeDtypeStruct((B_SC,), jnp.float32),
    compiler_params=pltpu.CompilerParams(needs_layout_passes=False),
    scratch_types=[
        pltpu.VMEM((BPW, 1), jnp.int32),
        pltpu.VMEM((BPW, 1), jnp.int32),
        [pltpu.VMEM((1, D), jnp.float32) for _ in range(NSLOT)],
        [pltpu.VMEM((1, D), jnp.float32) for _ in range(NSLOT)],
        pltpu.VMEM((BPW,), jnp.float32),
        pltpu.VMEM((L, L), jnp.float32),
        [pltpu.SemaphoreType.DMA for _ in range(NSLOT)],
        [pltpu.SemaphoreType.DMA for _ in range(NSLOT)],
    ],
)
def _slim_body(user_hbm, item_hbm, a_hbm, w_hbm, out_hbm,
               uidx, iidx, a_bufs, w_bufs, res, acc_buf, sems_a, sems_w):
    wid = lax.axis_index("s") * NC + lax.axis_index("c")
    pltpu.sync_copy(user_hbm.at[pl.ds(wid * BPW, BPW)], uidx)
    pltpu.sync_copy(item_hbm.at[pl.ds(wid * BPW, BPW)], iidx)

    lane_iota = lax.iota(jnp.int32, L)

    def start(e, slot):
        pltpu.async_copy(a_hbm.at[uidx.at[e]], a_bufs[slot], sems_a[slot])
        pltpu.async_copy(w_hbm.at[iidx.at[e]], w_bufs[slot], sems_w[slot])

    def wait(e, slot):
        pltpu.make_async_copy(
            a_hbm.at[uidx.at[e]], a_bufs[slot], sems_a[slot]).wait()
        pltpu.make_async_copy(
            w_hbm.at[iidx.at[e]], w_bufs[slot], sems_w[slot]).wait()

    for s in range(NSLOT):
        start(s, s)

    def group_body(g, carry):
        for c in range(L):
            slot = c % NSLOT
            e = g * L + c
            wait(e, slot)
            acc_buf[c] = _row_dot(a_bufs[slot], w_bufs[slot])

            @pl.when(e + NSLOT < BPW)
            def _():
                start(e + NSLOT, slot)

        totals = jnp.zeros((L,), jnp.float32)
        for c in range(L):
            col_idx = jnp.full((L,), c, jnp.int32)
            totals = totals + plsc.load_gather(acc_buf, [lane_iota, col_idx])
        res[pl.ds(g * L, L)] = totals
        return carry

    lax.fori_loop(0, BPW // L, group_body, 0)
    pltpu.sync_copy(res, out_hbm.at[pl.ds(wid * BPW, BPW)])


def _tc_body(uref, iref, a_hbm, w_hbm, out_ref, a_scr, w_scr, sems):
    strip_iota = lax.broadcasted_iota(jnp.int32, (STRIP,), 0)

    def start(e, s):
        pltpu.make_async_copy(
            a_hbm.at[uref[e]], a_scr.at[s], sems.at[0, s]).start()
        pltpu.make_async_copy(
            w_hbm.at[iref[e]], w_scr.at[s], sems.at[1, s]).start()

    def wait(e, s):
        pltpu.make_async_copy(
            a_hbm.at[uref[e]], a_scr.at[s], sems.at[0, s]).wait()
        pltpu.make_async_copy(
            w_hbm.at[iref[e]], w_scr.at[s], sems.at[1, s]).wait()

    for s in range(NBUF):
        start(s, s)

    gp_per_strip = STRIP // NBUF

    def group_body(g, acc):
        for s in range(NBUF):
            e = g * NBUF + s
            wait(e, s)
            prod = jnp.sum(a_scr[s] * w_scr[s])
            k = (g % gp_per_strip) * NBUF + s
            acc = jnp.where(strip_iota == k, prod, acc)

            @pl.when(e + NBUF < B_TC)
            def _():
                start(e + NBUF, s)

        @pl.when(g % gp_per_strip == gp_per_strip - 1)
        def _():
            out_ref[pl.ds((g // gp_per_strip) * STRIP, STRIP)] = acc

        return jnp.where(g % gp_per_strip == gp_per_strip - 1,
                         jnp.zeros((STRIP,), jnp.float32), acc)

    lax.fori_loop(0, B_TC // NBUF, group_body,
                  jnp.zeros((STRIP,), jnp.float32))


_tc_call = pl.pallas_call(
    _tc_body,
    grid_spec=pltpu.PrefetchScalarGridSpec(
        num_scalar_prefetch=2,
        grid=(1,),
        in_specs=[
            pl.BlockSpec(memory_space=pl.ANY),
            pl.BlockSpec(memory_space=pl.ANY),
        ],
        out_specs=pl.BlockSpec(memory_space=pltpu.MemorySpace.VMEM),
        scratch_shapes=[
            pltpu.VMEM((NBUF, D), jnp.float32),
            pltpu.VMEM((NBUF, D), jnp.float32),
            pltpu.SemaphoreType.DMA((2, NBUF)),
        ],
    ),
    out_shape=jax.ShapeDtypeStruct((B_TC,), jnp.float32),
)


def kernel(user, item, A, W):
    user = user.astype(jnp.int32)
    item = item.astype(jnp.int32)
    out_sc = _slim_body(user[:B_SC].reshape(B_SC, 1),
                        item[:B_SC].reshape(B_SC, 1), A, W)
    out_tc = _tc_call(user[B_SC:], item[B_SC:], A, W)
    return jnp.concatenate([out_sc, out_tc])


# speedup vs baseline: 7.4391x; 1.0665x over previous
import functools

import jax
import jax.numpy as jnp
from jax import lax
from jax.experimental import pallas as pl
from jax.experimental.pallas import tpu as pltpu
from jax.experimental.pallas import tpu_sc as plsc

B = 4096
D = 8192
L = 16
NC = 2
NS = 16
NW = NC * NS
BPW = B // NW
NSLOT = 4
UNROLL = 4

_mesh = plsc.VectorSubcoreMesh(core_axis_name="c", subcore_axis_name="s")


def _row_dot(a_ref, w_ref):
    def inner(j, accs):
        base = j * (4 * UNROLL * L)
        new = []
        for q in range(4):
            acc = accs[q]
            for u in range(UNROLL):
                off = base + (q * UNROLL + u) * L
                acc = acc + a_ref[0, pl.ds(off, L)] * w_ref[0, pl.ds(off, L)]
            new.append(acc)
        return tuple(new)

    zeros = jnp.zeros((L,), jnp.float32)
    accs = lax.fori_loop(0, D // (4 * UNROLL * L), inner,
                         (zeros, zeros, zeros, zeros))
    return (accs[0] + accs[1]) + (accs[2] + accs[3])


@functools.partial(
    pl.kernel,
    mesh=_mesh,
    out_type=jax.ShapeDtypeStruct((B,), jnp.float32),
    compiler_params=pltpu.CompilerParams(needs_layout_passes=False),
    scratch_types=[
        pltpu.VMEM((BPW, 1), jnp.int32),
        pltpu.VMEM((BPW, 1), jnp.int32),
        [pltpu.VMEM((1, D), jnp.float32) for _ in range(NSLOT)],
        [pltpu.VMEM((1, D), jnp.float32) for _ in range(NSLOT)],
        pltpu.VMEM((BPW,), jnp.float32),
        pltpu.VMEM((L, L), jnp.float32),
        [pltpu.SemaphoreType.DMA for _ in range(NSLOT)],
        [pltpu.SemaphoreType.DMA for _ in range(NSLOT)],
    ],
)
def _slim_body(user_hbm, item_hbm, a_hbm, w_hbm, out_hbm,
               uidx, iidx, a_bufs, w_bufs, res, acc_buf, sems_a, sems_w):
    wid = lax.axis_index("s") * NC + lax.axis_index("c")
    pltpu.sync_copy(user_hbm.at[pl.ds(wid * BPW, BPW)], uidx)
    pltpu.sync_copy(item_hbm.at[pl.ds(wid * BPW, BPW)], iidx)

    lane_iota = lax.iota(jnp.int32, L)

    def start(e, slot):
        pltpu.async_copy(a_hbm.at[uidx.at[e]], a_bufs[slot], sems_a[slot])
        pltpu.async_copy(w_hbm.at[iidx.at[e]], w_bufs[slot], sems_w[slot])

    def wait(e, slot):
        pltpu.make_async_copy(
            a_hbm.at[uidx.at[e]], a_bufs[slot], sems_a[slot]).wait()
        pltpu.make_async_copy(
            w_hbm.at[iidx.at[e]], w_bufs[slot], sems_w[slot]).wait()

    for s in range(NSLOT):
        start(s, s)

    def group_body(g, carry):
        for c in range(L):
            slot = c % NSLOT
            e = g * L + c
            wait(e, slot)
            acc_buf[c] = _row_dot(a_bufs[slot], w_bufs[slot])

            @pl.when(e + NSLOT < BPW)
            def _():
                start(e + NSLOT, slot)

        totals = jnp.zeros((L,), jnp.float32)
        for c in range(L):
            col_idx = jnp.full((L,), c, jnp.int32)
            totals = totals + plsc.load_gather(acc_buf, [lane_iota, col_idx])
        res[pl.ds(g * L, L)] = totals
        return carry

    lax.fori_loop(0, BPW // L, group_body, 0)
    pltpu.sync_copy(res, out_hbm.at[pl.ds(wid * BPW, BPW)])


def kernel(user, item, A, W):
    user2 = user.astype(jnp.int32).reshape(B, 1)
    item2 = item.astype(jnp.int32).reshape(B, 1)
    return _slim_body(user2, item2, A, W)
